# Initial kernel scaffold; baseline (speedup 1.0000x reference)
import jax
import jax.numpy as jnp
from jax.experimental import pallas as pl


def kernel(x, embedding_matrix, W1, b1, W2, b2, W3, b3):
    raise NotImplementedError("write your pallas kernel here")


# speedup vs baseline: 1.5869x; 1.5869x over previous
import functools

import jax
import jax.numpy as jnp
from jax import lax
from jax.experimental import pallas as pl
from jax.experimental.pallas import tpu as pltpu
from jax.experimental.pallas import tpu_sc as plsc

_NC = 2
_NS = 16
_NW = _NC * _NS
_LANES = 16


def _sc_gather_super(table8, x2):
    nw, npw = x2.shape
    mesh = plsc.VectorSubcoreMesh(core_axis_name="c", subcore_axis_name="s")

    @functools.partial(
        pl.kernel,
        mesh=mesh,
        compiler_params=pltpu.CompilerParams(use_tc_tiling_on_sc=False),
        out_type=jax.ShapeDtypeStruct((nw, npw, 8), jnp.float32),
        scratch_types=[
            pltpu.VMEM((npw,), jnp.int32),
            pltpu.VMEM((npw,), jnp.int32),
            pltpu.VMEM((npw, 8), jnp.float32),
            pltpu.SemaphoreType.DMA,
        ],
    )
    def gk(table_hbm, x_hbm, out_hbm, x_v, isup_v, rows_v, sem):
        wid = lax.axis_index("s") * _NC + lax.axis_index("c")
        pltpu.sync_copy(x_hbm.at[wid], x_v)

        def shift(k, carry):
            isup_v[pl.ds(k * _LANES, _LANES)] = x_v[pl.ds(k * _LANES, _LANES)] >> 2
            return carry

        lax.fori_loop(0, npw // _LANES, shift, 0)
        pltpu.async_copy(table_hbm.at[isup_v], rows_v, sem).wait()
        pltpu.sync_copy(rows_v, out_hbm.at[wid])

    return gk(table8, x2)


def _mlp_body(rows_ref, x_ref, w1_ref, b1_ref, w2_ref, b2_ref, w3_ref, b3_ref,
              out_ref):
    rows = rows_ref[...]
    low = x_ref[...] & 3
    e0 = jnp.zeros_like(low, dtype=jnp.float32)
    e1 = jnp.zeros_like(low, dtype=jnp.float32)
    for j in range(4):
        sel = low == j
        e0 = jnp.where(sel, rows[:, 2 * j:2 * j + 1], e0)
        e1 = jnp.where(sel, rows[:, 2 * j + 1:2 * j + 2], e1)
    h1 = jnp.maximum(e0 * w1_ref[0:1, :] + e1 * w1_ref[1:2, :] + b1_ref[...], 0.0)
    h2 = jnp.dot(h1, w2_ref[...], preferred_element_type=jnp.float32) + b2_ref[...]
    h2 = jnp.maximum(h2, 0.0)
    out_ref[...] = (
        jnp.dot(h2, w3_ref[...], preferred_element_type=jnp.float32) + b3_ref[...]
    )


def _mlp(rows, xflat, W1, b1, W2, b2, W3, b3, rows_per_block=2048):
    n = rows.shape[0]
    grid = n // rows_per_block
    h1d, h2d, od = W1.shape[1], W2.shape[1], W3.shape[1]
    return pl.pallas_call(
        _mlp_body,
        grid=(grid,),
        in_specs=[
            pl.BlockSpec((rows_per_block, 8), lambda i: (i, 0)),
            pl.BlockSpec((rows_per_block, 1), lambda i: (i, 0)),
            pl.BlockSpec((2, h1d), lambda i: (0, 0)),
            pl.BlockSpec((1, h1d), lambda i: (0, 0)),
            pl.BlockSpec((h1d, h2d), lambda i: (0, 0)),
            pl.BlockSpec((1, h2d), lambda i: (0, 0)),
            pl.BlockSpec((h2d, od), lambda i: (0, 0)),
            pl.BlockSpec((1, od), lambda i: (0, 0)),
        ],
        out_specs=pl.BlockSpec((rows_per_block, od), lambda i: (i, 0)),
        out_shape=jax.ShapeDtypeStruct((n, od), jnp.float32),
    )(
        rows,
        xflat,
        W1,
        b1.reshape(1, h1d),
        W2,
        b2.reshape(1, h2d),
        W3,
        b3.reshape(1, od),
    )


def kernel(x, embedding_matrix, W1, b1, W2, b2, W3, b3):
    bsz, seq = x.shape
    n = bsz * seq
    npw = n // _NW
    xi = x.astype(jnp.int32)
    table8 = embedding_matrix.reshape(embedding_matrix.shape[0] // 4, 8)
    rows = _sc_gather_super(table8, xi.reshape(_NW, npw)).reshape(n, 8)
    out = _mlp(rows, xi.reshape(n, 1), W1, b1, W2, b2, W3, b3)
    return out.reshape(bsz, seq, W3.shape[1])


# speedup vs baseline: 5.1339x; 3.2351x over previous
import functools

import jax
import jax.numpy as jnp
from jax import lax
from jax.experimental import pallas as pl
from jax.experimental.pallas import tpu as pltpu
from jax.experimental.pallas import tpu_sc as plsc

_NC = 2
_NS = 16
_NW = _NC * _NS
_LANES = 16


def _gather16(v, idx):
    return lax.gather(
        v,
        idx[:, None],
        dimension_numbers=lax.GatherDimensionNumbers(
            offset_dims=(), collapsed_slice_dims=(0,), start_index_map=(0,)),
        slice_sizes=(1,),
        mode=lax.GatherScatterMode.PROMISE_IN_BOUNDS,
    )


def _sc_gather_select(tableT16, x2):
    nw, npw = x2.shape
    rows_per_w = npw // 128
    chunk = npw // 4
    half = tableT16.shape[0] // 2
    mesh = plsc.VectorSubcoreMesh(core_axis_name="c", subcore_axis_name="s")

    @functools.partial(
        pl.kernel,
        mesh=mesh,
        compiler_params=pltpu.CompilerParams(use_tc_tiling_on_sc=False),
        out_type=(
            jax.ShapeDtypeStruct((nw * rows_per_w, 128), jnp.float32),
            jax.ShapeDtypeStruct((nw * rows_per_w, 128), jnp.float32),
        ),
        scratch_types=[
            pltpu.VMEM((npw,), jnp.int32),
            pltpu.VMEM((npw,), jnp.int32),
            pltpu.VMEM((npw,), jnp.int32),
            pltpu.VMEM((chunk, 16), jnp.float32),
            pltpu.VMEM((chunk, 16), jnp.float32),
            pltpu.VMEM((rows_per_w, 128), jnp.float32),
            pltpu.VMEM((rows_per_w, 128), jnp.float32),
            pltpu.SemaphoreType.DMA,
            pltpu.SemaphoreType.DMA,
        ],
    )
    def gk(table_hbm, x_hbm, out0_hbm, out1_hbm, x_v, is0_v, is1_v, r0_v,
           r1_v, e0_v, e1_v, sem0, sem1):
        wid = lax.axis_index("s") * _NC + lax.axis_index("c")
        pltpu.sync_copy(x_hbm.at[wid], x_v)

        def shift(k, carry):
            sl = pl.ds(k * _LANES, _LANES)
            xv = x_v[sl]
            s = xv >> 4
            is0_v[sl] = s
            is1_v[sl] = s + half
            x_v[sl] = xv & 15
            return carry

        lax.fori_loop(0, npw // _LANES, shift, 0)
        masks = [lax.iota(jnp.int32, _LANES) == l for l in range(1, _LANES)]

        for c in range(npw // chunk):
            cp0 = pltpu.async_copy(
                table_hbm.at[is0_v.at[pl.ds(c * chunk, chunk)]], r0_v, sem0)
            cp1 = pltpu.async_copy(
                table_hbm.at[is1_v.at[pl.ds(c * chunk, chunk)]], r1_v, sem1)
            cp0.wait()
            cp1.wait()

            def sel(k, carry, c=c):
                base = c * chunk + k * _LANES
                lo = x_v[pl.ds(base, _LANES)]
                acc0 = _gather16(r0_v[k * _LANES, :], lo)
                acc1 = _gather16(r1_v[k * _LANES, :], lo)
                for l in range(1, _LANES):
                    g0 = _gather16(r0_v[k * _LANES + l, :], lo)
                    g1 = _gather16(r1_v[k * _LANES + l, :], lo)
                    acc0 = jnp.where(masks[l - 1], g0, acc0)
                    acc1 = jnp.where(masks[l - 1], g1, acc1)
                row = base // 128
                lane = base % 128
                e0_v[row, pl.ds(lane, _LANES)] = acc0
                e1_v[row, pl.ds(lane, _LANES)] = acc1
                return carry

            lax.fori_loop(0, chunk // _LANES, sel, 0)

        pltpu.sync_copy(e0_v, out0_hbm.at[pl.ds(wid * rows_per_w, rows_per_w)])
        pltpu.sync_copy(e1_v, out1_hbm.at[pl.ds(wid * rows_per_w, rows_per_w)])

    return gk(tableT16, x2)


def _mlp_body(e0_ref, e1_ref, w1t_ref, b1_ref, w2t_ref, b2_ref, w3_ref,
              b3_ref, out_ref):
    w1c0 = w1t_ref[:, 0:1]
    w1c1 = w1t_ref[:, 1:2]
    b1c = b1_ref[...]
    w2t = w2t_ref[...]
    b2c = b2_ref[...]
    w3 = w3_ref[...]
    b3r = b3_ref[...]
    ngroups = e0_ref.shape[0]
    for g in range(ngroups):
        e0 = jnp.broadcast_to(e0_ref[g:g + 1, :], (128, 128))
        e1 = jnp.broadcast_to(e1_ref[g:g + 1, :], (128, 128))
        h1 = jnp.maximum(w1c0 * e0 + w1c1 * e1 + b1c, 0.0)
        h2 = jnp.dot(w2t, h1, preferred_element_type=jnp.float32) + b2c
        h2 = jnp.maximum(h2, 0.0)
        o = lax.dot_general(h2, w3, (((0,), (0,)), ((), ())),
                            preferred_element_type=jnp.float32)
        out_ref[pl.ds(g * 128, 128), :] = o + b3r


def _mlp(e0p, e1p, W1, b1, W2, b2, W3, b3, group_rows=16):
    nrows = e0p.shape[0]
    n = nrows * 128
    grid = nrows // group_rows
    h1d, h2d, od = W1.shape[1], W2.shape[1], W3.shape[1]
    return pl.pallas_call(
        _mlp_body,
        grid=(grid,),
        in_specs=[
            pl.BlockSpec((group_rows, 128), lambda i: (i, 0)),
            pl.BlockSpec((group_rows, 128), lambda i: (i, 0)),
            pl.BlockSpec((h1d, 2), lambda i: (0, 0)),
            pl.BlockSpec((h1d, 1), lambda i: (0, 0)),
            pl.BlockSpec((h2d, h1d), lambda i: (0, 0)),
            pl.BlockSpec((h2d, 1), lambda i: (0, 0)),
            pl.BlockSpec((h2d, od), lambda i: (0, 0)),
            pl.BlockSpec((1, od), lambda i: (0, 0)),
        ],
        out_specs=pl.BlockSpec((group_rows * 128, od), lambda i: (i, 0)),
        out_shape=jax.ShapeDtypeStruct((n, od), jnp.float32),
    )(
        e0p,
        e1p,
        W1.T,
        b1.reshape(h1d, 1),
        W2.T,
        b2.reshape(h2d, 1),
        W3,
        b3.reshape(1, od),
    )


def kernel(x, embedding_matrix, W1, b1, W2, b2, W3, b3):
    bsz, seq = x.shape
    n = bsz * seq
    npw = n // _NW
    xi = x.astype(jnp.int32)
    tableT16 = embedding_matrix.T.reshape(embedding_matrix.shape[0] // 8, 16)
    e0p, e1p = _sc_gather_select(tableT16, xi.reshape(_NW, npw))
    out = _mlp(e0p, e1p, W1, b1, W2, b2, W3, b3)
    return out.reshape(bsz, seq, W3.shape[1])


# speedup vs baseline: 7.0190x; 1.3672x over previous
import functools

import jax
import jax.numpy as jnp
from jax import lax
from jax.experimental import pallas as pl
from jax.experimental.pallas import tpu as pltpu
from jax.experimental.pallas import tpu_sc as plsc

_NC = 2
_NS = 16
_NW = _NC * _NS
_LANES = 16


def _gather16(v, idx):
    return lax.gather(
        v,
        idx[:, None],
        dimension_numbers=lax.GatherDimensionNumbers(
            offset_dims=(), collapsed_slice_dims=(0,), start_index_map=(0,)),
        slice_sizes=(1,),
        mode=lax.GatherScatterMode.PROMISE_IN_BOUNDS,
    )


def _sc_gather_select(tableT16, x2):
    nw, npw = x2.shape
    rows_per_w = npw // 128
    chunk = npw // 4
    half = tableT16.shape[0] // 2
    mesh = plsc.VectorSubcoreMesh(core_axis_name="c", subcore_axis_name="s")

    @functools.partial(
        pl.kernel,
        mesh=mesh,
        compiler_params=pltpu.CompilerParams(use_tc_tiling_on_sc=False),
        out_type=(
            jax.ShapeDtypeStruct((nw * rows_per_w, 128), jnp.float32),
            jax.ShapeDtypeStruct((nw * rows_per_w, 128), jnp.float32),
        ),
        scratch_types=[
            pltpu.VMEM((npw,), jnp.int32),
            pltpu.VMEM((npw,), jnp.int32),
            pltpu.VMEM((npw,), jnp.int32),
            pltpu.VMEM((chunk, 16), jnp.float32),
            pltpu.VMEM((chunk, 16), jnp.float32),
            pltpu.VMEM((rows_per_w, 128), jnp.float32),
            pltpu.VMEM((rows_per_w, 128), jnp.float32),
            pltpu.SemaphoreType.DMA,
            pltpu.SemaphoreType.DMA,
        ],
    )
    def gk(table_hbm, x_hbm, out0_hbm, out1_hbm, x_v, is0_v, is1_v, r0_v,
           r1_v, e0_v, e1_v, sem0, sem1):
        wid = lax.axis_index("s") * _NC + lax.axis_index("c")
        pltpu.sync_copy(x_hbm.at[wid], x_v)

        def shift(k, carry):
            sl = pl.ds(k * _LANES, _LANES)
            xv = x_v[sl]
            s = xv >> 4
            is0_v[sl] = s
            is1_v[sl] = s + half
            x_v[sl] = xv & 15
            return carry

        lax.fori_loop(0, npw // _LANES, shift, 0)
        masks = [lax.iota(jnp.int32, _LANES) == l for l in range(1, _LANES)]

        for c in range(npw // chunk):
            cp0 = pltpu.async_copy(
                table_hbm.at[is0_v.at[pl.ds(c * chunk, chunk)]], r0_v, sem0)
            cp1 = pltpu.async_copy(
                table_hbm.at[is1_v.at[pl.ds(c * chunk, chunk)]], r1_v, sem1)
            cp0.wait()
            cp1.wait()

            def sel(k, carry, c=c):
                base = c * chunk + k * _LANES
                lo = x_v[pl.ds(base, _LANES)]
                acc0 = _gather16(r0_v[k * _LANES, :], lo)
                acc1 = _gather16(r1_v[k * _LANES, :], lo)
                for l in range(1, _LANES):
                    g0 = _gather16(r0_v[k * _LANES + l, :], lo)
                    g1 = _gather16(r1_v[k * _LANES + l, :], lo)
                    acc0 = jnp.where(masks[l - 1], g0, acc0)
                    acc1 = jnp.where(masks[l - 1], g1, acc1)
                row = base // 128
                lane = base % 128
                e0_v[row, pl.ds(lane, _LANES)] = acc0
                e1_v[row, pl.ds(lane, _LANES)] = acc1
                return carry

            lax.fori_loop(0, chunk // _LANES, sel, 0)

        pltpu.sync_copy(e0_v, out0_hbm.at[pl.ds(wid * rows_per_w, rows_per_w)])
        pltpu.sync_copy(e1_v, out1_hbm.at[pl.ds(wid * rows_per_w, rows_per_w)])

    return gk(tableT16, x2)


def _mlp_body(e0_ref, e1_ref, w1t_ref, b1_ref, w2t_ref, b2_ref, w3_ref,
              b3_ref, out_ref):
    w1t = w1t_ref[...]
    b1c = b1_ref[...]
    w2t = w2t_ref[...]
    b2c = b2_ref[...]
    w3 = w3_ref[...]
    b3c = b3_ref[...]
    nseq, ngroups = e0_ref.shape[0], e0_ref.shape[1]
    for s in range(nseq):
        e0s = e0_ref[s]
        e1s = e1_ref[s]
        for g in range(ngroups):
            e = jnp.concatenate([e0s[g:g + 1, :], e1s[g:g + 1, :]], axis=0)
            h1 = jnp.dot(w1t, e, preferred_element_type=jnp.float32) + b1c
            h1 = jnp.maximum(h1, 0.0)
            h2 = jnp.dot(w2t, h1, preferred_element_type=jnp.float32) + b2c
            h2 = jnp.maximum(h2, 0.0)
            o = lax.dot_general(w3, h2, (((0,), (0,)), ((), ())),
                                preferred_element_type=jnp.float32)
            out_ref[:, s, pl.ds(g * 128, 128)] = o + b3c


def _mlp(e0p, e1p, W1, b1, W2, b2, W3, b3, bsz, seq, seq_blk=8, b_blk=2048):
    rows_per_seq = bsz // 128
    groups_per_blk = b_blk // 128
    e3 = e0p.reshape(seq, rows_per_seq, 128)
    f3 = e1p.reshape(seq, rows_per_seq, 128)
    grid = (pl.cdiv(seq, seq_blk), bsz // b_blk)
    h1d, h2d, od = W1.shape[1], W2.shape[1], W3.shape[1]
    return pl.pallas_call(
        _mlp_body,
        grid=grid,
        in_specs=[
            pl.BlockSpec((seq_blk, groups_per_blk, 128), lambda k, j: (k, j, 0)),
            pl.BlockSpec((seq_blk, groups_per_blk, 128), lambda k, j: (k, j, 0)),
            pl.BlockSpec((h1d, 2), lambda k, j: (0, 0)),
            pl.BlockSpec((h1d, 1), lambda k, j: (0, 0)),
            pl.BlockSpec((h2d, h1d), lambda k, j: (0, 0)),
            pl.BlockSpec((h2d, 1), lambda k, j: (0, 0)),
            pl.BlockSpec((h2d, od), lambda k, j: (0, 0)),
            pl.BlockSpec((od, 1), lambda k, j: (0, 0)),
        ],
        out_specs=pl.BlockSpec(
            (od, seq_blk, b_blk), lambda k, j: (0, k, j)),
        out_shape=jax.ShapeDtypeStruct((od, seq, bsz), jnp.float32),
    )(
        e3,
        f3,
        W1.T,
        b1.reshape(h1d, 1),
        W2.T,
        b2.reshape(h2d, 1),
        W3,
        b3.reshape(od, 1),
    )


def kernel(x, embedding_matrix, W1, b1, W2, b2, W3, b3):
    bsz, seq = x.shape
    n = bsz * seq
    npw = n // _NW
    xi = jnp.transpose(x.astype(jnp.int32))
    tableT16 = embedding_matrix.T.reshape(embedding_matrix.shape[0] // 8, 16)
    e0p, e1p = _sc_gather_select(tableT16, xi.reshape(_NW, npw))
    out = _mlp(e0p, e1p, W1, b1, W2, b2, W3, b3, bsz, seq)
    return jnp.transpose(out, (2, 1, 0))


# speedup vs baseline: 25.4870x; 3.6311x over previous
import functools

import jax
import jax.numpy as jnp
from jax import lax
from jax.experimental import pallas as pl
from jax.experimental.pallas import tpu as pltpu
from jax.experimental.pallas import tpu_sc as plsc

_NC = 2
_NS = 16
_NW = _NC * _NS
_LANES = 16


def _gather16(v, idx):
    return lax.gather(
        v,
        idx[:, None],
        dimension_numbers=lax.GatherDimensionNumbers(
            offset_dims=(), collapsed_slice_dims=(0,), start_index_map=(0,)),
        slice_sizes=(1,),
        mode=lax.GatherScatterMode.PROMISE_IN_BOUNDS,
    )


def _sc_gather_select(tableT16, x2):
    nw, npw = x2.shape
    rows_per_w = npw // 128
    chunk = npw // 4
    half = tableT16.shape[0] // 2
    mesh = plsc.VectorSubcoreMesh(core_axis_name="c", subcore_axis_name="s")

    @functools.partial(
        pl.kernel,
        mesh=mesh,
        compiler_params=pltpu.CompilerParams(use_tc_tiling_on_sc=False),
        out_type=(
            jax.ShapeDtypeStruct((nw * rows_per_w, 128), jnp.float32),
            jax.ShapeDtypeStruct((nw * rows_per_w, 128), jnp.float32),
        ),
        scratch_types=[
            pltpu.VMEM((npw,), jnp.int32),
            pltpu.VMEM((npw,), jnp.int32),
            pltpu.VMEM((npw,), jnp.int32),
            pltpu.VMEM((chunk, 16), jnp.float32),
            pltpu.VMEM((chunk, 16), jnp.float32),
            pltpu.VMEM((rows_per_w, 128), jnp.float32),
            pltpu.VMEM((rows_per_w, 128), jnp.float32),
            pltpu.SemaphoreType.DMA,
            pltpu.SemaphoreType.DMA,
        ],
    )
    def gk(table_hbm, x_hbm, out0_hbm, out1_hbm, x_v, is0_v, is1_v, r0_v,
           r1_v, e0_v, e1_v, sem0, sem1):
        wid = lax.axis_index("s") * _NC + lax.axis_index("c")
        pltpu.sync_copy(x_hbm.at[wid], x_v)

        def shift(k, carry):
            sl = pl.ds(k * _LANES, _LANES)
            xv = x_v[sl]
            s = xv >> 4
            is0_v[sl] = s
            is1_v[sl] = s + half
            x_v[sl] = xv & 15
            return carry

        lax.fori_loop(0, npw // _LANES, shift, 0)
        masks = [lax.iota(jnp.int32, _LANES) == l for l in range(1, _LANES)]

        for c in range(npw // chunk):
            cp0 = pltpu.async_copy(
                table_hbm.at[is0_v.at[pl.ds(c * chunk, chunk)]], r0_v, sem0)
            cp1 = pltpu.async_copy(
                table_hbm.at[is1_v.at[pl.ds(c * chunk, chunk)]], r1_v, sem1)
            cp0.wait()
            cp1.wait()

            def sel(k, carry, c=c):
                base = c * chunk + k * _LANES
                lo = x_v[pl.ds(base, _LANES)]
                acc0 = _gather16(r0_v[k * _LANES, :], lo)
                acc1 = _gather16(r1_v[k * _LANES, :], lo)
                for l in range(1, _LANES):
                    g0 = _gather16(r0_v[k * _LANES + l, :], lo)
                    g1 = _gather16(r1_v[k * _LANES + l, :], lo)
                    acc0 = jnp.where(masks[l - 1], g0, acc0)
                    acc1 = jnp.where(masks[l - 1], g1, acc1)
                row = base // 128
                lane = base % 128
                e0_v[row, pl.ds(lane, _LANES)] = acc0
                e1_v[row, pl.ds(lane, _LANES)] = acc1
                return carry

            lax.fori_loop(0, chunk // _LANES, sel, 0)

        pltpu.sync_copy(e0_v, out0_hbm.at[pl.ds(wid * rows_per_w, rows_per_w)])
        pltpu.sync_copy(e1_v, out1_hbm.at[pl.ds(wid * rows_per_w, rows_per_w)])

    return gk(tableT16, x2)


def _mlp_body(e0_ref, e1_ref, w1t_ref, b1_ref, w2t_ref, b2_ref, w3_ref,
              b3_ref, out_ref, e_ref):
    w1t = w1t_ref[...]
    b1c = b1_ref[...]
    w2t = w2t_ref[...]
    b2c = b2_ref[...]
    w3 = w3_ref[...]
    b3c = b3_ref[...]
    nseq, ngroups = e0_ref.shape[0], e0_ref.shape[1]
    for s in range(nseq):
        e0s = e0_ref[s]
        e1s = e1_ref[s]
        for g in range(ngroups):
            e_ref[0:1, g * 128:(g + 1) * 128] = e0s[g:g + 1, :]
            e_ref[1:2, g * 128:(g + 1) * 128] = e1s[g:g + 1, :]
        e = e_ref[...]
        h1 = jnp.dot(w1t, e, preferred_element_type=jnp.float32) + b1c
        h1 = jnp.maximum(h1, 0.0)
        h2 = jnp.dot(w2t, h1, preferred_element_type=jnp.float32) + b2c
        h2 = jnp.maximum(h2, 0.0)
        o = lax.dot_general(w3, h2, (((0,), (0,)), ((), ())),
                            preferred_element_type=jnp.float32)
        out_ref[:, s, :] = o + b3c


def _mlp(e0p, e1p, W1, b1, W2, b2, W3, b3, bsz, seq, seq_blk=8, b_blk=2048):
    rows_per_seq = bsz // 128
    groups_per_blk = b_blk // 128
    e3 = e0p.reshape(seq, rows_per_seq, 128)
    f3 = e1p.reshape(seq, rows_per_seq, 128)
    grid = (pl.cdiv(seq, seq_blk), bsz // b_blk)
    h1d, h2d, od = W1.shape[1], W2.shape[1], W3.shape[1]
    return pl.pallas_call(
        _mlp_body,
        grid=grid,
        in_specs=[
            pl.BlockSpec((seq_blk, groups_per_blk, 128), lambda k, j: (k, j, 0)),
            pl.BlockSpec((seq_blk, groups_per_blk, 128), lambda k, j: (k, j, 0)),
            pl.BlockSpec((h1d, 2), lambda k, j: (0, 0)),
            pl.BlockSpec((h1d, 1), lambda k, j: (0, 0)),
            pl.BlockSpec((h2d, h1d), lambda k, j: (0, 0)),
            pl.BlockSpec((h2d, 1), lambda k, j: (0, 0)),
            pl.BlockSpec((h2d, od), lambda k, j: (0, 0)),
            pl.BlockSpec((od, 1), lambda k, j: (0, 0)),
        ],
        out_specs=pl.BlockSpec(
            (od, seq_blk, b_blk), lambda k, j: (0, k, j)),
        out_shape=jax.ShapeDtypeStruct((od, seq, bsz), jnp.float32),
        scratch_shapes=[pltpu.VMEM((2, b_blk), jnp.float32)],
    )(
        e3,
        f3,
        W1.T,
        b1.reshape(h1d, 1),
        W2.T,
        b2.reshape(h2d, 1),
        W3,
        b3.reshape(od, 1),
    )


def kernel(x, embedding_matrix, W1, b1, W2, b2, W3, b3):
    bsz, seq = x.shape
    n = bsz * seq
    npw = n // _NW
    xi = jnp.transpose(x.astype(jnp.int32))
    tableT16 = embedding_matrix.T.reshape(embedding_matrix.shape[0] // 8, 16)
    e0p, e1p = _sc_gather_select(tableT16, xi.reshape(_NW, npw))
    out = _mlp(e0p, e1p, W1, b1, W2, b2, W3, b3, bsz, seq)
    return jnp.transpose(out, (2, 1, 0))
